# Initial kernel scaffold; baseline (speedup 1.0000x reference)
import jax
import jax.numpy as jnp
from jax.experimental import pallas as pl


def kernel(x, edge_index, edge_type, node_type, W0, etab0, Wr0, al0, ar0, ae0, W1, etab1, Wr1, al1, ar1, ae1, Wres1, bres1):
    raise NotImplementedError("write your pallas kernel here")


# speedup vs baseline: 23.4925x; 23.4925x over previous
import functools

import jax
import jax.numpy as jnp
from jax import lax
from jax.experimental import pallas as pl
from jax.experimental.pallas import tpu as pltpu
from jax.experimental.pallas import tpu_sc as plsc

N = 10000
E = 320000
IN_DIM = 128
HIDDEN = 64
NUM_CLASSES = 16
EDGE_DIM = 16
NUM_ETYPES = 5
H0 = 8
BETA = 0.05
SLOPE = 0.2

NB = 400
NBLK = N // NB
EB = 400
NROWS = N // 32 * 2

f32 = jnp.float32
i32 = jnp.int32


_SC_PARAMS = pltpu.CompilerParams(needs_layout_passes=False,
                                  use_tc_tiling_on_sc=False)


def _mesh():
    return plsc.VectorSubcoreMesh(core_axis_name="c", subcore_axis_name="s")


def _iota16():
    return lax.iota(i32, 16)


def _tc_tab_body(etab0, Wr0, AE0, etab1, Wr1, AE1, tab0, tab1m):
    rows0 = []
    rows1 = []
    E0 = etab0[...]
    E1 = etab1[...]
    z18 = jnp.zeros((1, 8), f32)
    for t in range(NUM_ETYPES):
        f0 = jnp.dot(E0[t:t + 1, :], Wr0[t],
                     preferred_element_type=f32)
        he0 = jnp.dot(f0, AE0[...], preferred_element_type=f32)
        rows0.append(jnp.concatenate([he0, z18], axis=1))
        f1 = jnp.dot(E1[t:t + 1, :], Wr1[t],
                     preferred_element_type=f32)
        he1 = jnp.dot(f1, AE1[...], preferred_element_type=f32)
        rows1.append(jnp.broadcast_to(he1, (1, 16)))
    for _ in range(8 - NUM_ETYPES):
        rows0.append(jnp.zeros((1, 16), f32))
        rows1.append(jnp.zeros((1, 16), f32))
    tab0[...] = jnp.concatenate(rows0, axis=0)
    tab1m[...] = jnp.concatenate(rows1, axis=0)


def _tc1_body(x, W0, AL, AR, emb0h, t0s, t0d):
    emb = jnp.dot(x[...], W0[...], preferred_element_type=f32)
    for h in range(H0):
        emb0h[h] = emb[:, h * HIDDEN:(h + 1) * HIDDEN]
    hl = jnp.dot(emb, AL[...], preferred_element_type=f32)
    hr = jnp.dot(emb, AR[...], preferred_element_type=f32)
    z = jnp.zeros((NB, 8), f32)
    t0s[...] = jnp.concatenate([hl, z], axis=1)
    t0d[...] = jnp.concatenate([hr, z], axis=1)


def _tc2_body(agg, s0p, W1p, Wr1p, b1, al1v, ar1v,
              emb1, hl1, hr1, s0t0, res1):
    s0 = s0p[0] + s0p[1]
    s0g = jnp.maximum(s0, 1e-30)
    segs = []
    for h in range(H0):
        segs.append(agg[h] / s0g[:, h:h + 1])
    hpre = jnp.concatenate(segs, axis=1)
    h1 = jnp.where(hpre > 0, hpre, jnp.exp(jnp.minimum(hpre, 0.0)) - 1.0)
    e1 = jnp.dot(h1, W1p[...], preferred_element_type=f32)
    emb1[...] = e1
    res1[...] = jnp.dot(h1, Wr1p[...], preferred_element_type=f32) + b1[...]
    hl1[...] = jnp.sum(e1 * al1v[...], axis=1, keepdims=True)
    hr1[...] = jnp.sum(e1 * ar1v[...], axis=1, keepdims=True)
    s0t0[...] = s0[:, 0:1]


def _tcs1_body(s1p, s1tot):
    s1tot[...] = s1p[0, :, 0:1] + s1p[1, :, 0:1]


def _tc3_body(a1p, res1, out):
    out[...] = jax.nn.sigmoid(a1p[0] + a1p[1] + res1[...])


def _p0a_body(src, dst, et, t0s, t0d, tab0,
              e0r, e00, s0p,
              srcv, dstv, etv, srow, drow, rows, e00b, tabv, zb, s0_sp):
    c = lax.axis_index("c")
    s = lax.axis_index("s")
    iot = _iota16()
    lane0 = iot == 0
    lanelt8 = iot < 8
    zero16 = jnp.zeros((16,), f32)

    def zloop(i, _):
        zb[i, :] = zero16
        return 0
    lax.fori_loop(0, NROWS, zloop, 0)
    pltpu.sync_copy(tab0, tabv)
    pltpu.sync_copy(zb, s0_sp.at[pl.ds(s * NROWS, NROWS)])
    plsc.subcore_barrier()

    nblk = (E // 2) // EB // 16
    base0 = c * (E // 2) + s * ((E // 2) // 16)

    def blk(b, _):
        base = base0 + b * EB
        pltpu.sync_copy(src.at[pl.ds(base, EB)], srcv)
        pltpu.sync_copy(dst.at[pl.ds(base, EB)], dstv)
        pltpu.sync_copy(et.at[pl.ds(base, EB)], etv)
        pltpu.sync_copy(t0s.at[srcv], srow)
        pltpu.sync_copy(t0d.at[dstv], drow)

        def edge(j, _):
            jf = jnp.full((16,), j, i32)
            etl = plsc.load_gather(etv, [jf])
            he = plsc.load_gather(tabv, [etl, iot])
            a = srow[j, :] + drow[j, :] + he
            attv = jnp.maximum(a, SLOPE * a)
            ev = jnp.where(lanelt8, jnp.exp(attv), 0.0)
            rows[j, :] = ev
            plsc.store_scatter(e00b, [jf], ev, mask=lane0)
            return 0
        lax.fori_loop(0, EB, edge, 0)

        pltpu.sync_copy(rows, s0_sp.at[dstv], add=True)
        pltpu.sync_copy(rows, e0r.at[pl.ds(base, EB)])
        pltpu.sync_copy(e00b, e00.at[pl.ds(base, EB)])
        return 0
    lax.fori_loop(0, nblk, blk, 0)

    plsc.subcore_barrier()
    pltpu.sync_copy(s0_sp.at[pl.ds(s * NROWS, NROWS)],
                    s0p.at[pl.ds(c * N + s * NROWS, NROWS)])


def _p0m_body(src, dst, e0r, emb0f,
              agg0f,
              srcv, dstv, gidx, rows, e0rows, zb, a_sp):
    c = lax.axis_index("c")
    s = lax.axis_index("s")
    zero16 = jnp.zeros((16,), f32)

    def zloop(i, _):
        for k in range(4):
            zb[i, pl.ds(16 * k, 16)] = zero16
        return 0
    lax.fori_loop(0, 125, zloop, 0)

    nblk = (E // 16) // EB
    base0 = s * (E // 16)

    for jh in range(4):
        h = 4 * c + jh
        for z in range(5):
            pltpu.sync_copy(zb, a_sp.at[pl.ds(s * NROWS + z * 125, 125)])
        plsc.subcore_barrier()

        def blk(b, _):
            base = base0 + b * EB
            pltpu.sync_copy(src.at[pl.ds(base, EB)], srcv)
            pltpu.sync_copy(dst.at[pl.ds(base, EB)], dstv)
            pltpu.sync_copy(e0r.at[pl.ds(base, EB)], e0rows)
            hN = h * N

            def gfix(i, _):
                gidx[pl.ds(16 * i, 16)] = srcv[pl.ds(16 * i, 16)] + hN
                return 0
            lax.fori_loop(0, EB // 16, gfix, 0)
            pltpu.sync_copy(emb0f.at[gidx], rows)

            hf = jnp.full((16,), h, i32)

            def edge(j, _):
                jf = jnp.full((16,), j, i32)
                av = plsc.load_gather(e0rows, [jf, hf])
                for k in range(4):
                    rows[j, pl.ds(16 * k, 16)] = rows[j, pl.ds(16 * k, 16)] * av
                return 0
            lax.fori_loop(0, EB, edge, 0)

            pltpu.sync_copy(rows, a_sp.at[dstv], add=True)
            return 0
        lax.fori_loop(0, nblk, blk, 0)

        plsc.subcore_barrier()
        pltpu.sync_copy(a_sp.at[pl.ds(s * NROWS, NROWS)],
                        agg0f.at[pl.ds(h * N + s * NROWS, NROWS)])


def _p1a_body(src, dst, et, hl1, hr1, tab1m,
              e1, s1p,
              srcv, dstv, etv, e1b, rows1, hlv, hrv, tabv, zb, s1_sp):
    c = lax.axis_index("c")
    s = lax.axis_index("s")
    iot = _iota16()
    zero16 = jnp.zeros((16,), f32)
    z16i = jnp.zeros((16,), i32)

    def zloop(i, _):
        zb[i, :] = zero16
        return 0
    lax.fori_loop(0, NROWS, zloop, 0)

    def zrows(i, _):
        rows1[i, :] = zero16
        return 0
    lax.fori_loop(0, EB, zrows, 0)

    pltpu.sync_copy(hl1, hlv)
    pltpu.sync_copy(hr1, hrv)
    pltpu.sync_copy(tab1m, tabv)
    pltpu.sync_copy(zb, s1_sp.at[pl.ds(s * NROWS, NROWS)])
    plsc.subcore_barrier()

    nblk = (E // 2) // EB // 16
    base0 = c * (E // 2) + s * ((E // 2) // 16)

    def blk(b, _):
        base = base0 + b * EB
        pltpu.sync_copy(src.at[pl.ds(base, EB)], srcv)
        pltpu.sync_copy(dst.at[pl.ds(base, EB)], dstv)
        pltpu.sync_copy(et.at[pl.ds(base, EB)], etv)

        def vle(v, _):
            sv = srcv[pl.ds(16 * v, 16)]
            dv = dstv[pl.ds(16 * v, 16)]
            ev = etv[pl.ds(16 * v, 16)]
            hlg = plsc.load_gather(hlv, [sv])
            hrg = plsc.load_gather(hrv, [dv])
            heg = plsc.load_gather(tabv, [ev, z16i])
            a = hlg + hrg + heg
            attv = jnp.maximum(a, SLOPE * a)
            e1v = jnp.exp(attv)
            e1b[pl.ds(16 * v, 16)] = e1v
            plsc.store_scatter(rows1, [jnp.full((16,), 16 * v, i32) + iot, z16i],
                               e1v)
            return 0
        lax.fori_loop(0, EB // 16, vle, 0)

        pltpu.sync_copy(rows1, s1_sp.at[dstv], add=True)
        pltpu.sync_copy(e1b, e1.at[pl.ds(base, EB)])
        return 0
    lax.fori_loop(0, nblk, blk, 0)

    plsc.subcore_barrier()
    pltpu.sync_copy(s1_sp.at[pl.ds(s * NROWS, NROWS)],
                    s1p.at[pl.ds(c * N + s * NROWS, NROWS)])


def _p1m_body(src, dst, e1, e00, s1tot, s0t0, emb1,
              a1p,
              srcv, dstv, e1v, e00v, blendb, rows, s1v, s0v, zb, a_sp):
    c = lax.axis_index("c")
    s = lax.axis_index("s")
    zero16 = jnp.zeros((16,), f32)

    def zloop(i, _):
        zb[i, :] = zero16
        return 0
    lax.fori_loop(0, NROWS, zloop, 0)
    pltpu.sync_copy(s1tot, s1v)
    pltpu.sync_copy(s0t0, s0v)
    pltpu.sync_copy(zb, a_sp.at[pl.ds(s * NROWS, NROWS)])
    plsc.subcore_barrier()

    nblk = (E // 2) // EB // 16
    base0 = c * (E // 2) + s * ((E // 2) // 16)

    def blk(b, _):
        base = base0 + b * EB
        pltpu.sync_copy(src.at[pl.ds(base, EB)], srcv)
        pltpu.sync_copy(dst.at[pl.ds(base, EB)], dstv)
        pltpu.sync_copy(e1.at[pl.ds(base, EB)], e1v)
        pltpu.sync_copy(e00.at[pl.ds(base, EB)], e00v)
        pltpu.sync_copy(emb1.at[srcv], rows)

        def vle(v, _):
            dv = dstv[pl.ds(16 * v, 16)]
            s1g = plsc.load_gather(s1v, [dv])
            s0g = plsc.load_gather(s0v, [dv])
            bl = ((1.0 - BETA) * e1v[pl.ds(16 * v, 16)] / s1g
                  + BETA * e00v[pl.ds(16 * v, 16)] / s0g)
            blendb[pl.ds(16 * v, 16)] = bl
            return 0
        lax.fori_loop(0, EB // 16, vle, 0)

        def edge(j, _):
            jf = jnp.full((16,), j, i32)
            bv = plsc.load_gather(blendb, [jf])
            rows[j, :] = rows[j, :] * bv
            return 0
        lax.fori_loop(0, EB, edge, 0)

        pltpu.sync_copy(rows, a_sp.at[dstv], add=True)
        return 0
    lax.fori_loop(0, nblk, blk, 0)

    plsc.subcore_barrier()
    pltpu.sync_copy(a_sp.at[pl.ds(s * NROWS, NROWS)],
                    a1p.at[pl.ds(c * N + s * NROWS, NROWS)])


def _sds(shape, dtype=f32):
    return jax.ShapeDtypeStruct(shape, dtype)


def kernel(x, edge_index, edge_type, node_type, W0, etab0, Wr0, al0, ar0, ae0,
           W1, etab1, Wr1, al1, ar1, ae1, Wres1, bres1):
    src = edge_index[0]
    dst = edge_index[1]
    et = edge_type

    eye8 = jnp.eye(H0, dtype=f32)
    AL = jnp.einsum("hd,hg->hdg", al0[0], eye8).reshape(H0 * HIDDEN, H0)
    AR = jnp.einsum("hd,hg->hdg", ar0[0], eye8).reshape(H0 * HIDDEN, H0)
    AE0 = jnp.einsum("hd,hg->hdg", ae0[0], eye8).reshape(H0 * EDGE_DIM, H0)
    AE1 = ae1.reshape(EDGE_DIM, 1)
    W1p = W1.reshape(HIDDEN, H0, NUM_CLASSES).transpose(1, 0, 2).reshape(
        H0 * HIDDEN, NUM_CLASSES)
    Wr1p = Wres1.reshape(HIDDEN, H0, NUM_CLASSES).transpose(1, 0, 2).reshape(
        H0 * HIDDEN, NUM_CLASSES)
    b1 = bres1.reshape(1, NUM_CLASSES)
    al1v = al1.reshape(1, NUM_CLASSES)
    ar1v = ar1.reshape(1, NUM_CLASSES)

    tab0, tab1m = pl.pallas_call(
        _tc_tab_body,
        out_shape=[_sds((8, 16)), _sds((8, 16))],
    )(etab0, Wr0, AE0, etab1, Wr1, AE1)

    emb0h, t0s, t0d = pl.pallas_call(
        _tc1_body,
        grid=(NBLK,),
        in_specs=[
            pl.BlockSpec((NB, IN_DIM), lambda b: (b, 0)),
            pl.BlockSpec((IN_DIM, H0 * HIDDEN), lambda b: (0, 0)),
            pl.BlockSpec((H0 * HIDDEN, H0), lambda b: (0, 0)),
            pl.BlockSpec((H0 * HIDDEN, H0), lambda b: (0, 0)),
        ],
        out_specs=[
            pl.BlockSpec((H0, NB, HIDDEN), lambda b: (0, b, 0)),
            pl.BlockSpec((NB, 16), lambda b: (b, 0)),
            pl.BlockSpec((NB, 16), lambda b: (b, 0)),
        ],
        out_shape=[_sds((H0, N, HIDDEN)), _sds((N, 16)), _sds((N, 16))],
    )(x, W0, AL, AR)

    mesh = _mesh()
    e0r, e00, s0p = pl.kernel(
        _p0a_body, mesh=mesh, compiler_params=_SC_PARAMS,
        out_type=[_sds((E, 16)), _sds((E,)), _sds((2 * N, 16))],
        scratch_types=[
            pltpu.VMEM((EB,), i32), pltpu.VMEM((EB,), i32),
            pltpu.VMEM((EB,), i32),
            pltpu.VMEM((EB, 16), f32), pltpu.VMEM((EB, 16), f32),
            pltpu.VMEM((EB, 16), f32), pltpu.VMEM((EB,), f32),
            pltpu.VMEM((8, 16), f32), pltpu.VMEM((NROWS, 16), f32),
            pltpu.VMEM_SHARED((N, 16), f32),
        ],
    )(src, dst, et, t0s, t0d, tab0)

    agg0f, = pl.kernel(
        _p0m_body, mesh=mesh, compiler_params=_SC_PARAMS,
        out_type=[_sds((H0 * N, HIDDEN))],
        scratch_types=[
            pltpu.VMEM((EB,), i32), pltpu.VMEM((EB,), i32),
            pltpu.VMEM((EB,), i32),
            pltpu.VMEM((EB, HIDDEN), f32), pltpu.VMEM((EB, 16), f32),
            pltpu.VMEM((125, HIDDEN), f32),
            pltpu.VMEM_SHARED((N, HIDDEN), f32),
        ],
    )(src, dst, e0r, emb0h.reshape(H0 * N, HIDDEN))

    emb1, hl1, hr1, s0t0, res1 = pl.pallas_call(
        _tc2_body,
        grid=(NBLK,),
        in_specs=[
            pl.BlockSpec((H0, NB, HIDDEN), lambda b: (0, b, 0)),
            pl.BlockSpec((2, NB, 16), lambda b: (0, b, 0)),
            pl.BlockSpec((H0 * HIDDEN, NUM_CLASSES), lambda b: (0, 0)),
            pl.BlockSpec((H0 * HIDDEN, NUM_CLASSES), lambda b: (0, 0)),
            pl.BlockSpec((1, NUM_CLASSES), lambda b: (0, 0)),
            pl.BlockSpec((1, NUM_CLASSES), lambda b: (0, 0)),
            pl.BlockSpec((1, NUM_CLASSES), lambda b: (0, 0)),
        ],
        out_specs=[
            pl.BlockSpec((NB, NUM_CLASSES), lambda b: (b, 0)),
            pl.BlockSpec((NB, 1), lambda b: (b, 0)),
            pl.BlockSpec((NB, 1), lambda b: (b, 0)),
            pl.BlockSpec((NB, 1), lambda b: (b, 0)),
            pl.BlockSpec((NB, NUM_CLASSES), lambda b: (b, 0)),
        ],
        out_shape=[_sds((N, NUM_CLASSES)), _sds((N, 1)), _sds((N, 1)),
                   _sds((N, 1)), _sds((N, NUM_CLASSES))],
    )(agg0f.reshape(H0, N, HIDDEN), s0p.reshape(2, N, 16),
      W1p, Wr1p, b1, al1v, ar1v)

    e1, s1p = pl.kernel(
        _p1a_body, mesh=mesh, compiler_params=_SC_PARAMS,
        out_type=[_sds((E,)), _sds((2 * N, 16))],
        scratch_types=[
            pltpu.VMEM((EB,), i32), pltpu.VMEM((EB,), i32),
            pltpu.VMEM((EB,), i32),
            pltpu.VMEM((EB,), f32), pltpu.VMEM((EB, 16), f32),
            pltpu.VMEM((N,), f32), pltpu.VMEM((N,), f32),
            pltpu.VMEM((8, 16), f32), pltpu.VMEM((NROWS, 16), f32),
            pltpu.VMEM_SHARED((N, 16), f32),
        ],
    )(src, dst, et, hl1.reshape(N), hr1.reshape(N), tab1m)

    s1tot, = pl.pallas_call(
        _tcs1_body,
        grid=(NBLK,),
        in_specs=[pl.BlockSpec((2, NB, 16), lambda b: (0, b, 0))],
        out_specs=[pl.BlockSpec((NB, 1), lambda b: (b, 0))],
        out_shape=[_sds((N, 1))],
    )(s1p.reshape(2, N, 16))

    a1p, = pl.kernel(
        _p1m_body, mesh=mesh, compiler_params=_SC_PARAMS,
        out_type=[_sds((2 * N, 16))],
        scratch_types=[
            pltpu.VMEM((EB,), i32), pltpu.VMEM((EB,), i32),
            pltpu.VMEM((EB,), f32), pltpu.VMEM((EB,), f32),
            pltpu.VMEM((EB,), f32), pltpu.VMEM((EB, 16), f32),
            pltpu.VMEM((N,), f32), pltpu.VMEM((N,), f32),
            pltpu.VMEM((NROWS, 16), f32),
            pltpu.VMEM_SHARED((N, 16), f32),
        ],
    )(src, dst, e1, e00, s1tot.reshape(N), s0t0.reshape(N), emb1)

    out, = pl.pallas_call(
        _tc3_body,
        grid=(NBLK,),
        in_specs=[
            pl.BlockSpec((2, NB, 16), lambda b: (0, b, 0)),
            pl.BlockSpec((NB, 16), lambda b: (b, 0)),
        ],
        out_specs=[pl.BlockSpec((NB, 16), lambda b: (b, 0))],
        out_shape=[_sds((N, NUM_CLASSES))],
    )(a1p.reshape(2, N, 16), res1)

    return out
